# baseline (speedup 1.0000x reference)
import functools

import jax
import jax.numpy as jnp
from jax.experimental import pallas as pl
from jax.experimental.pallas import tpu as pltpu

EPS = 1e-7
BIG = 3e38


def _fused_body(xyz1t_ref, p1_ref, w1a_ref, w1b_ref, w2_ref, b1_ref, b2_ref,
                xyz2_ref, p2_ref, out_ref, pw_scratch):
    j = pl.program_id(1)

    @pl.when(j == 0)
    def _():
        pw_scratch[...] = jnp.dot(p1_ref[0], w1a_ref[...],
                                  preferred_element_type=jnp.float32
                                  ).astype(jnp.bfloat16)

    x1t = xyz1t_ref[0]
    sq1 = jnp.sum(x1t * x1t, axis=0, keepdims=True)
    x2 = xyz2_ref[0]
    sq2 = jnp.sum(x2 * x2, axis=1, keepdims=True)
    inner = jnp.dot(x2, x1t, preferred_element_type=jnp.float32)
    t = sq1 - 2.0 * inner

    m1 = jnp.min(t, axis=1, keepdims=True)
    e1 = t == m1
    t2 = jnp.where(e1, BIG, t)
    m2 = jnp.min(t2, axis=1, keepdims=True)
    e2 = t2 == m2
    t3 = jnp.where(e2, BIG, t2)
    m3 = jnp.min(t3, axis=1, keepdims=True)
    e3 = t3 == m3

    d = [jnp.maximum(v + sq2, EPS) for v in (m1, m2, m3)]
    w = [1.0 / dk for dk in d]
    norm = w[0] + w[1] + w[2]
    w = [wk / norm for wk in w]

    zero = jnp.zeros_like(t)
    A = jnp.where(e1, w[0],
                  jnp.where(e2, w[1],
                            jnp.where(e3, w[2], zero))
                  ).astype(jnp.bfloat16)

    h1 = jnp.dot(A, pw_scratch[...], preferred_element_type=jnp.float32)
    h1 += jnp.dot(p2_ref[0], w1b_ref[...], preferred_element_type=jnp.float32)
    h1 = jnp.maximum(h1 + b1_ref[...], 0.0).astype(jnp.bfloat16)
    h2 = jnp.dot(h1, w2_ref[...], preferred_element_type=jnp.float32)
    h2 = jnp.maximum(h2 + b2_ref[...], 0.0)
    out_ref[0] = jnp.concatenate([x2, h2], axis=1)


@functools.partial(jax.jit, static_argnames=("blk",))
def _fused(xyz1t, points1, xyz2, points2, W1a, W1b, W2, b1, b2, blk=1024):
    B, N2, _ = points2.shape
    N1 = points1.shape[1]
    C1 = points1.shape[2]
    grid = (B, N2 // blk)
    return pl.pallas_call(
        _fused_body,
        grid=grid,
        in_specs=[
            pl.BlockSpec((1, 3, N1), lambda b, j: (b, 0, 0)),
            pl.BlockSpec((1, N1, C1), lambda b, j: (b, 0, 0)),
            pl.BlockSpec((C1, 512), lambda b, j: (0, 0)),
            pl.BlockSpec((256, 512), lambda b, j: (0, 0)),
            pl.BlockSpec((512, 512), lambda b, j: (0, 0)),
            pl.BlockSpec((1, 512), lambda b, j: (0, 0)),
            pl.BlockSpec((1, 512), lambda b, j: (0, 0)),
            pl.BlockSpec((1, blk, 3), lambda b, j: (b, j, 0)),
            pl.BlockSpec((1, blk, 256), lambda b, j: (b, j, 0)),
        ],
        out_specs=pl.BlockSpec((1, blk, 515), lambda b, j: (b, j, 0)),
        out_shape=jax.ShapeDtypeStruct((B, N2, 515), jnp.float32),
        scratch_shapes=[pltpu.VMEM((N1, 512), jnp.bfloat16)],
        compiler_params=pltpu.CompilerParams(
            dimension_semantics=("arbitrary", "arbitrary"),
        ),
    )(xyz1t, points1, W1a, W1b, W2, b1, b2, xyz2, points2)


def kernel(inputs_0, inputs_1, W1, b1, W2, b2):
    xyz1 = inputs_0[:, :, 0:3]
    points1 = inputs_0[:, :, 3:]
    xyz2 = inputs_1[:, :, 0:3]
    points2 = inputs_1[:, :, 3:]
    xyz1t = jnp.transpose(xyz1, (0, 2, 1))
    W1a = W1[:512, :].astype(jnp.bfloat16)
    W1b = W1[512:, :].astype(jnp.bfloat16)
    new_points = _fused(xyz1t, points1.astype(jnp.bfloat16),
                        xyz2, points2.astype(jnp.bfloat16),
                        W1a, W1b, W2.astype(jnp.bfloat16),
                        b1.reshape(1, -1), b2.reshape(1, -1))
    return (new_points, xyz2)


# speedup vs baseline: 1.0150x; 1.0150x over previous
import functools

import jax
import jax.numpy as jnp
from jax.experimental import pallas as pl
from jax.experimental.pallas import tpu as pltpu

EPS = 1e-7
BIG = 3e38


def _fused_body(xyz1t_ref, p1_ref, w1a_ref, w1b_ref, w2_ref, b1_ref, b2_ref,
                xyz2_ref, p2_ref, out_ref, pw_scratch):
    j = pl.program_id(1)

    @pl.when(j == 0)
    def _():
        pw_scratch[...] = jnp.dot(p1_ref[0], w1a_ref[...],
                                  preferred_element_type=jnp.float32
                                  ).astype(jnp.bfloat16)

    x1t = xyz1t_ref[0]
    sq1 = jnp.sum(x1t * x1t, axis=0, keepdims=True)
    x2 = xyz2_ref[0]
    sq2 = jnp.sum(x2 * x2, axis=1, keepdims=True)
    inner = jnp.dot(x2, x1t, preferred_element_type=jnp.float32)
    t = sq1 - 2.0 * inner

    m1 = jnp.min(t, axis=1, keepdims=True)
    t2 = jnp.where(t == m1, BIG, t)
    m2 = jnp.min(t2, axis=1, keepdims=True)
    m3 = jnp.min(jnp.where(t2 == m2, BIG, t2), axis=1, keepdims=True)

    d = [jnp.maximum(v + sq2, EPS) for v in (m1, m2, m3)]
    w = [1.0 / dk for dk in d]
    norm = w[0] + w[1] + w[2]
    w = [wk / norm for wk in w]

    zero = jnp.zeros_like(t)
    A = jnp.where(t == m1, w[0],
                  jnp.where(t == m2, w[1],
                            jnp.where(t == m3, w[2], zero))
                  ).astype(jnp.bfloat16)

    h1 = jnp.dot(A, pw_scratch[...], preferred_element_type=jnp.float32)
    h1 += jnp.dot(p2_ref[0], w1b_ref[...], preferred_element_type=jnp.float32)
    h1 = jnp.maximum(h1 + b1_ref[...], 0.0).astype(jnp.bfloat16)
    h2 = jnp.dot(h1, w2_ref[...], preferred_element_type=jnp.float32)
    h2 = jnp.maximum(h2 + b2_ref[...], 0.0)
    out_ref[0] = jnp.concatenate([x2, h2], axis=1)


@functools.partial(jax.jit, static_argnames=("blk",))
def _fused(xyz1t, points1, xyz2, points2, W1a, W1b, W2, b1, b2, blk=1024):
    B, N2, _ = points2.shape
    N1 = points1.shape[1]
    C1 = points1.shape[2]
    grid = (B, N2 // blk)
    return pl.pallas_call(
        _fused_body,
        grid=grid,
        in_specs=[
            pl.BlockSpec((1, 3, N1), lambda b, j: (b, 0, 0)),
            pl.BlockSpec((1, N1, C1), lambda b, j: (b, 0, 0)),
            pl.BlockSpec((C1, 512), lambda b, j: (0, 0)),
            pl.BlockSpec((256, 512), lambda b, j: (0, 0)),
            pl.BlockSpec((512, 512), lambda b, j: (0, 0)),
            pl.BlockSpec((1, 512), lambda b, j: (0, 0)),
            pl.BlockSpec((1, 512), lambda b, j: (0, 0)),
            pl.BlockSpec((1, blk, 3), lambda b, j: (b, j, 0)),
            pl.BlockSpec((1, blk, 256), lambda b, j: (b, j, 0)),
        ],
        out_specs=pl.BlockSpec((1, blk, 515), lambda b, j: (b, j, 0)),
        out_shape=jax.ShapeDtypeStruct((B, N2, 515), jnp.float32),
        scratch_shapes=[pltpu.VMEM((N1, 512), jnp.bfloat16)],
        compiler_params=pltpu.CompilerParams(
            dimension_semantics=("arbitrary", "arbitrary"),
        ),
    )(xyz1t, points1, W1a, W1b, W2, b1, b2, xyz2, points2)


def kernel(inputs_0, inputs_1, W1, b1, W2, b2):
    xyz1 = inputs_0[:, :, 0:3]
    points1 = inputs_0[:, :, 3:]
    xyz2 = inputs_1[:, :, 0:3]
    points2 = inputs_1[:, :, 3:]
    xyz1t = jnp.transpose(xyz1, (0, 2, 1))
    W1a = W1[:512, :].astype(jnp.bfloat16)
    W1b = W1[512:, :].astype(jnp.bfloat16)
    new_points = _fused(xyz1t, points1.astype(jnp.bfloat16),
                        xyz2, points2.astype(jnp.bfloat16),
                        W1a, W1b, W2.astype(jnp.bfloat16),
                        b1.reshape(1, -1), b2.reshape(1, -1))
    return (new_points, xyz2)
